# baseline (speedup 1.0000x reference)
import jax
import jax.numpy as jnp
from jax.experimental import pallas as pl
from jax.experimental.pallas import tpu as pltpu


def _band_selector(n_in, n_pool, dtype):
    kw = jnp.arange(5)[:, None, None, None]
    w = jnp.arange(n_in)[None, :, None, None]
    cw = jnp.arange(2)[None, None, :, None]
    s = jnp.arange(n_pool)[None, None, None, :]
    return (w == 2 * s + cw + kw).astype(dtype)


def _conv1_mat(c1w):
    w4 = c1w[:75, :16].reshape(5, 5, 3, 16)
    sel = _band_selector(32, 14, c1w.dtype)
    m = jnp.einsum('hkco,kwzs->hcwzso', w4, sel)
    m = m.reshape(5, 96, 2, 224)
    m = jnp.pad(m, ((0, 0), (0, 0), (0, 0), (0, 32)))
    return m.reshape(480, 512)


def _conv2_mat(c2w):
    w4 = c2w[:400, :32].reshape(5, 5, 16, 32)
    sel = _band_selector(14, 5, c2w.dtype)
    m = jnp.einsum('hkco,kwzs->hwczso', w4, sel)
    m = m.reshape(5, 224, 2, 160)
    m = jnp.pad(m, ((0, 0), (0, 32), (0, 0), (0, 0)))
    return m.reshape(1280, 320)


def _fc1_mat(f1w):
    m = f1w.reshape(5, 5, 128, 128)[:, :, :32, :].reshape(5, 160, 128)
    return m.reshape(800, 128)


def _lenet_kernel(x_ref, w1_ref, b1_ref, w2_ref, b2_ref, f1_ref, f1b_ref,
                  f2_ref, f2b_ref, f3_ref, f3b_ref, o_ref, y1_s, y2_s):
    w1 = w1_ref[...]
    for r in range(14):
        rows = [x_ref[2 * r + t][:, :96] for t in range(6)]
        lhs_a = jnp.concatenate(rows[:5], axis=1)
        lhs_b = jnp.concatenate(rows[1:], axis=1)
        ya = jnp.dot(lhs_a, w1, preferred_element_type=jnp.float32)
        yb = jnp.dot(lhs_b, w1, preferred_element_type=jnp.float32)
        y = jnp.maximum(ya, yb)
        y = jnp.maximum(y[:, :256], y[:, 256:]) + b1_ref[...]
        y1_s[r] = jnp.maximum(y, 0.0).astype(jnp.bfloat16)

    w2 = w2_ref[...]
    for r in range(5):
        lhs = jnp.concatenate([y1_s[2 * r + t] for t in range(6)], axis=1)
        ya = jnp.dot(lhs[:, :1280], w2, preferred_element_type=jnp.float32)
        yb = jnp.dot(lhs[:, 256:], w2, preferred_element_type=jnp.float32)
        y = jnp.maximum(ya, yb)
        y = jnp.maximum(y[:, :160], y[:, 160:]) + b2_ref[...]
        y2_s[r] = jnp.maximum(y, 0.0).astype(jnp.bfloat16)

    feat = jnp.concatenate([y2_s[r] for r in range(5)], axis=1)
    h = jnp.dot(feat, f1_ref[...], preferred_element_type=jnp.float32)
    h = jnp.maximum(h + f1b_ref[...], 0.0).astype(jnp.bfloat16)
    h = jnp.dot(h, f2_ref[...], preferred_element_type=jnp.float32)
    h = jnp.maximum(h + f2b_ref[...], 0.0).astype(jnp.bfloat16)
    o_ref[...] = (jnp.dot(h, f3_ref[...], preferred_element_type=jnp.float32)
                  + f3b_ref[...])


def kernel(c1w, c1b, c2w, c2b, f1w, f1b, f2w, f2b, f3w, f3b, x_nchw):
    n = x_nchw.shape[0]
    bsz = 512 if n % 512 == 0 else n

    xt = jnp.transpose(x_nchw, (2, 0, 1, 3)).reshape(32, n, 96)
    xt = jnp.pad(xt, ((0, 0), (0, 0), (0, 32))).astype(jnp.bfloat16)

    w1 = _conv1_mat(c1w).astype(jnp.bfloat16)
    w2 = _conv2_mat(c2w).astype(jnp.bfloat16)
    f1 = _fc1_mat(f1w).astype(jnp.bfloat16)
    b1 = jnp.pad(jnp.tile(c1b[:, :16], (1, 14)), ((0, 0), (0, 32)))
    b2 = jnp.tile(c2b[:, :32], (1, 5))

    out = pl.pallas_call(
        _lenet_kernel,
        out_shape=jax.ShapeDtypeStruct((n, 128), jnp.float32),
        grid=(n // bsz,),
        in_specs=[
            pl.BlockSpec((32, bsz, 128), lambda b: (0, b, 0)),
            pl.BlockSpec((480, 512), lambda b: (0, 0)),
            pl.BlockSpec((1, 256), lambda b: (0, 0)),
            pl.BlockSpec((1280, 320), lambda b: (0, 0)),
            pl.BlockSpec((1, 160), lambda b: (0, 0)),
            pl.BlockSpec((800, 128), lambda b: (0, 0)),
            pl.BlockSpec((1, 128), lambda b: (0, 0)),
            pl.BlockSpec((128, 128), lambda b: (0, 0)),
            pl.BlockSpec((1, 128), lambda b: (0, 0)),
            pl.BlockSpec((128, 128), lambda b: (0, 0)),
            pl.BlockSpec((1, 128), lambda b: (0, 0)),
        ],
        out_specs=pl.BlockSpec((bsz, 128), lambda b: (b, 0)),
        scratch_shapes=[
            pltpu.VMEM((14, bsz, 256), jnp.bfloat16),
            pltpu.VMEM((5, bsz, 160), jnp.bfloat16),
        ],
        compiler_params=pltpu.CompilerParams(
            dimension_semantics=("parallel",)),
    )(xt, w1, b1, w2, b2, f1, f1b,
      f2w.astype(jnp.bfloat16), f2b, f3w.astype(jnp.bfloat16), f3b)
    return out[:, :10]


# speedup vs baseline: 1.0156x; 1.0156x over previous
import jax
import jax.numpy as jnp
from jax.experimental import pallas as pl
from jax.experimental.pallas import tpu as pltpu


def _band_selector(n_in, n_pool, dtype):
    kw = jnp.arange(5)[:, None, None, None]
    w = jnp.arange(n_in)[None, :, None, None]
    cw = jnp.arange(2)[None, None, :, None]
    s = jnp.arange(n_pool)[None, None, None, :]
    return (w == 2 * s + cw + kw).astype(dtype)


def _conv1_mat(c1w):
    w4 = c1w[:75, :16].reshape(5, 5, 3, 16)
    sel = _band_selector(32, 14, c1w.dtype)
    m = jnp.einsum('hkco,kwzs->hcwzso', w4, sel)
    m = m.reshape(5, 96, 2, 224)
    m = jnp.pad(m, ((0, 0), (0, 0), (0, 0), (0, 32)))
    return m.reshape(480, 512)


def _conv2_mat(c2w):
    w4 = c2w[:400, :32].reshape(5, 5, 16, 32)
    sel = _band_selector(14, 5, c2w.dtype)
    m = jnp.einsum('hkco,kwzs->hwczso', w4, sel)
    m = m.reshape(5, 224, 2, 160)
    m = jnp.pad(m, ((0, 0), (0, 32), (0, 0), (0, 96)))
    return m.reshape(1280, 512)


def _fc1_mat(f1w):
    m = f1w.reshape(5, 5, 128, 128)[:, :, :32, :].reshape(5, 160, 128)
    m = jnp.pad(m, ((0, 0), (0, 96), (0, 0)))
    return m.reshape(1280, 128)


def _lenet_kernel(x_ref, w1_ref, b1_ref, w2_ref, b2_ref, f1_ref, f1b_ref,
                  f2_ref, f2b_ref, f3_ref, f3b_ref, o_ref, y1_s, y2_s):
    w1 = w1_ref[...]
    for r in range(14):
        rows = [x_ref[2 * r + t][:, :96] for t in range(6)]
        lhs_a = jnp.concatenate(rows[:5], axis=1)
        lhs_b = jnp.concatenate(rows[1:], axis=1)
        ya = jnp.dot(lhs_a, w1, preferred_element_type=jnp.float32)
        yb = jnp.dot(lhs_b, w1, preferred_element_type=jnp.float32)
        y = jnp.maximum(ya, yb)
        y = jnp.maximum(y[:, :256], y[:, 256:]) + b1_ref[...]
        y1_s[r] = jnp.maximum(y, 0.0).astype(jnp.bfloat16)

    w2 = w2_ref[...]
    for r in range(5):
        lhs = jnp.concatenate([y1_s[2 * r + t] for t in range(6)], axis=1)
        ya = jnp.dot(lhs[:, :1280], w2, preferred_element_type=jnp.float32)
        yb = jnp.dot(lhs[:, 256:], w2, preferred_element_type=jnp.float32)
        y = jnp.maximum(ya, yb)
        y = jnp.maximum(y[:, :256], y[:, 256:]) + b2_ref[...]
        y2_s[r] = jnp.maximum(y, 0.0).astype(jnp.bfloat16)

    feat = jnp.concatenate([y2_s[r] for r in range(5)], axis=1)
    h = jnp.dot(feat, f1_ref[...], preferred_element_type=jnp.float32)
    h = jnp.maximum(h + f1b_ref[...], 0.0).astype(jnp.bfloat16)
    h = jnp.dot(h, f2_ref[...], preferred_element_type=jnp.float32)
    h = jnp.maximum(h + f2b_ref[...], 0.0).astype(jnp.bfloat16)
    o_ref[...] = (jnp.dot(h, f3_ref[...], preferred_element_type=jnp.float32)
                  + f3b_ref[...])


def kernel(c1w, c1b, c2w, c2b, f1w, f1b, f2w, f2b, f3w, f3b, x_nchw):
    n = x_nchw.shape[0]
    bsz = 512 if n % 512 == 0 else n

    xt = jnp.transpose(x_nchw, (2, 0, 1, 3)).reshape(32, n, 96)
    xt = jnp.pad(xt, ((0, 0), (0, 0), (0, 32))).astype(jnp.bfloat16)

    w1 = _conv1_mat(c1w).astype(jnp.bfloat16)
    w2 = _conv2_mat(c2w).astype(jnp.bfloat16)
    f1 = _fc1_mat(f1w).astype(jnp.bfloat16)
    b1 = jnp.pad(jnp.tile(c1b[:, :16], (1, 14)), ((0, 0), (0, 32)))
    b2 = jnp.pad(jnp.tile(c2b[:, :32], (1, 5)), ((0, 0), (0, 96)))

    out = pl.pallas_call(
        _lenet_kernel,
        out_shape=jax.ShapeDtypeStruct((n, 128), jnp.float32),
        grid=(n // bsz,),
        in_specs=[
            pl.BlockSpec((32, bsz, 128), lambda b: (0, b, 0)),
            pl.BlockSpec((480, 512), lambda b: (0, 0)),
            pl.BlockSpec((1, 256), lambda b: (0, 0)),
            pl.BlockSpec((1280, 512), lambda b: (0, 0)),
            pl.BlockSpec((1, 256), lambda b: (0, 0)),
            pl.BlockSpec((1280, 128), lambda b: (0, 0)),
            pl.BlockSpec((1, 128), lambda b: (0, 0)),
            pl.BlockSpec((128, 128), lambda b: (0, 0)),
            pl.BlockSpec((1, 128), lambda b: (0, 0)),
            pl.BlockSpec((128, 128), lambda b: (0, 0)),
            pl.BlockSpec((1, 128), lambda b: (0, 0)),
        ],
        out_specs=pl.BlockSpec((bsz, 128), lambda b: (b, 0)),
        scratch_shapes=[
            pltpu.VMEM((14, bsz, 256), jnp.bfloat16),
            pltpu.VMEM((5, bsz, 256), jnp.bfloat16),
        ],
        compiler_params=pltpu.CompilerParams(
            dimension_semantics=("parallel",)),
    )(xt, w1, b1, w2, b2, f1, f1b,
      f2w.astype(jnp.bfloat16), f2b, f3w.astype(jnp.bfloat16), f3b)
    return out[:, :10]
